# Initial kernel scaffold; baseline (speedup 1.0000x reference)
import jax
import jax.numpy as jnp
from jax.experimental import pallas as pl


def kernel(x, edge_index, idx, W0, b0, W1, b1, Wm, bm):
    raise NotImplementedError("write your pallas kernel here")


# speedup vs baseline: 30.5780x; 30.5780x over previous
import functools

import jax
import jax.numpy as jnp
from jax import lax
from jax.experimental import pallas as pl
from jax.experimental.pallas import tpu as pltpu
from jax.experimental.pallas import tpu_sc as plsc

N_NODES = 10000
N_EDGES = 320000
IN_CH = 128
HID = 64
OUT_CH = 5
N_IDX = 2048

NC = 2
NS = 16
NW = NC * NS
CH = 128
NP = 10240
ROWS_PER_TILE = NP // NS
CPW = 80
E_PAD = NW * CH * CPW
DEGW = 16
MBLK = 2048

_mesh = plsc.VectorSubcoreMesh(core_axis_name="c", subcore_axis_name="s")
_f32 = jnp.float32


@functools.partial(
    pl.kernel,
    out_type=jax.ShapeDtypeStruct((NC, NP, DEGW), _f32),
    mesh=_mesh,
    compiler_params=pltpu.CompilerParams(use_tc_tiling_on_sc=False),
    scratch_types=[
        pltpu.VMEM((CPW, CH), jnp.int32),
        pltpu.VMEM((CH, DEGW), _f32),
        pltpu.VMEM_SHARED((NP, DEGW), _f32),
        pltpu.SemaphoreType.DMA,
    ],
)
def _deg_kernel(dst_hbm, ones_hbm, zeros_hbm, out_hbm, dst_v, ones_v, acc_sp, sem):
    c = lax.axis_index("c")
    s = lax.axis_index("s")
    wid = c * NS + s
    pltpu.sync_copy(ones_hbm, ones_v)
    pltpu.sync_copy(dst_hbm.at[pl.ds(wid * CPW, CPW)], dst_v)
    row0 = s * ROWS_PER_TILE
    pltpu.sync_copy(zeros_hbm.at[pl.ds(row0, ROWS_PER_TILE)],
                    acc_sp.at[pl.ds(row0, ROWS_PER_TILE)])
    plsc.subcore_barrier()

    def body(j, carry):
        pltpu.sync_copy(ones_v, acc_sp.at[dst_v.at[j]], add=True)
        return carry

    lax.fori_loop(0, CPW, body, 0)
    plsc.subcore_barrier()
    pltpu.sync_copy(acc_sp.at[pl.ds(row0, ROWS_PER_TILE)],
                    out_hbm.at[c].at[pl.ds(row0, ROWS_PER_TILE)])


@functools.partial(
    pl.kernel,
    out_type=jax.ShapeDtypeStruct((NC, NP, HID), _f32),
    mesh=_mesh,
    compiler_params=pltpu.CompilerParams(use_tc_tiling_on_sc=False),
    scratch_types=[
        pltpu.VMEM((CPW, CH), jnp.int32),
        pltpu.VMEM((CPW, CH), jnp.int32),
        pltpu.VMEM((CH, HID), _f32),
        pltpu.VMEM((CH, HID), _f32),
        pltpu.VMEM_SHARED((NP, HID), _f32),
        pltpu.SemaphoreType.DMA,
        pltpu.SemaphoreType.DMA,
    ],
)
def _agg_kernel(ht_hbm, src_hbm, dst_hbm, zeros_hbm, out_hbm,
                src_v, dst_v, rows_a, rows_b, acc_sp, sem_a, sem_b):
    c = lax.axis_index("c")
    s = lax.axis_index("s")
    wid = c * NS + s
    pltpu.sync_copy(src_hbm.at[pl.ds(wid * CPW, CPW)], src_v)
    pltpu.sync_copy(dst_hbm.at[pl.ds(wid * CPW, CPW)], dst_v)
    row0 = s * ROWS_PER_TILE
    pltpu.sync_copy(zeros_hbm.at[pl.ds(row0, ROWS_PER_TILE)],
                    acc_sp.at[pl.ds(row0, ROWS_PER_TILE)])
    plsc.subcore_barrier()

    pltpu.async_copy(ht_hbm.at[src_v.at[0]], rows_a, sem_a)

    def body(i, carry):
        j = 2 * i

        pltpu.make_async_copy(ht_hbm.at[src_v.at[j]], rows_a, sem_a).wait()
        pltpu.async_copy(ht_hbm.at[src_v.at[j + 1]], rows_b, sem_b)
        pltpu.sync_copy(rows_a, acc_sp.at[dst_v.at[j]], add=True)

        pltpu.make_async_copy(ht_hbm.at[src_v.at[j + 1]], rows_b, sem_b).wait()

        @pl.when(j + 2 < CPW)
        def _():
            pltpu.async_copy(ht_hbm.at[src_v.at[j + 2]], rows_a, sem_a)

        pltpu.sync_copy(rows_b, acc_sp.at[dst_v.at[j + 1]], add=True)
        return carry

    lax.fori_loop(0, CPW // 2, body, 0)

    plsc.subcore_barrier()
    pltpu.sync_copy(acc_sp.at[pl.ds(row0, ROWS_PER_TILE)],
                    out_hbm.at[c].at[pl.ds(row0, ROWS_PER_TILE)])


_SEL_PER_W = N_IDX // NW


@functools.partial(
    pl.kernel,
    out_type=(
        jax.ShapeDtypeStruct((N_IDX, HID), _f32),
        jax.ShapeDtypeStruct((N_IDX, HID), _f32),
        jax.ShapeDtypeStruct((N_IDX, HID), _f32),
        jax.ShapeDtypeStruct((N_IDX, DEGW), _f32),
    ),
    mesh=_mesh,
    compiler_params=pltpu.CompilerParams(use_tc_tiling_on_sc=False),
    scratch_types=[
        pltpu.VMEM((_SEL_PER_W,), jnp.int32),
        pltpu.VMEM((_SEL_PER_W, HID), _f32),
        pltpu.VMEM((_SEL_PER_W, DEGW), _f32),
        pltpu.SemaphoreType.DMA,
    ],
)
def _sel_kernel(acc_a_hbm, acc_b_hbm, ht_hbm, dis_hbm, idx_hbm,
                o_a, o_b, o_h, o_d, idx_v, buf_h, buf_d, sem):
    c = lax.axis_index("c")
    s = lax.axis_index("s")
    wid = c * NS + s
    base = wid * _SEL_PER_W
    pltpu.sync_copy(idx_hbm.at[pl.ds(base, _SEL_PER_W)], idx_v)
    pltpu.async_copy(acc_a_hbm.at[idx_v], buf_h, sem).wait()
    pltpu.sync_copy(buf_h, o_a.at[pl.ds(base, _SEL_PER_W)])
    pltpu.async_copy(acc_b_hbm.at[idx_v], buf_h, sem).wait()
    pltpu.sync_copy(buf_h, o_b.at[pl.ds(base, _SEL_PER_W)])
    pltpu.async_copy(ht_hbm.at[idx_v], buf_h, sem).wait()
    pltpu.sync_copy(buf_h, o_h.at[pl.ds(base, _SEL_PER_W)])
    pltpu.async_copy(dis_hbm.at[idx_v], buf_d, sem).wait()
    pltpu.sync_copy(buf_d, o_d.at[pl.ds(base, _SEL_PER_W)])


def _tcA_body(x_ref, w0_ref, degp_ref, ht_ref, dis_ref):
    deg = degp_ref[0] + degp_ref[1] + 1.0
    dis = lax.rsqrt(deg)
    h = jnp.dot(x_ref[...], w0_ref[...], preferred_element_type=_f32)
    ht_ref[...] = h * dis[:, 0:1]
    dis_ref[...] = dis


_tcA = pl.pallas_call(
    _tcA_body,
    grid=(NP // MBLK,),
    in_specs=[
        pl.BlockSpec((MBLK, IN_CH), lambda i: (i, 0)),
        pl.BlockSpec((IN_CH, HID), lambda i: (0, 0)),
        pl.BlockSpec((NC, MBLK, DEGW), lambda i: (0, i, 0)),
    ],
    out_specs=[
        pl.BlockSpec((MBLK, HID), lambda i: (i, 0)),
        pl.BlockSpec((MBLK, DEGW), lambda i: (i, 0)),
    ],
    out_shape=[
        jax.ShapeDtypeStruct((NP, HID), _f32),
        jax.ShapeDtypeStruct((NP, DEGW), _f32),
    ],
)


def _tcB_body(accp_ref, ht0_ref, dis_ref, w1_ref, b0_ref, ht1_ref):
    dis = dis_ref[:, 0:1]
    pre = (accp_ref[0] + accp_ref[1] + ht0_ref[...]) * dis + b0_ref[...]
    h1 = jnp.where(pre >= 0, pre, 0.01 * pre)
    ht1_ref[...] = jnp.dot(h1, w1_ref[...], preferred_element_type=_f32) * dis


_tcB = pl.pallas_call(
    _tcB_body,
    grid=(NP // MBLK,),
    in_specs=[
        pl.BlockSpec((NC, MBLK, HID), lambda i: (0, i, 0)),
        pl.BlockSpec((MBLK, HID), lambda i: (i, 0)),
        pl.BlockSpec((MBLK, DEGW), lambda i: (i, 0)),
        pl.BlockSpec((HID, HID), lambda i: (0, 0)),
        pl.BlockSpec((1, HID), lambda i: (0, 0)),
    ],
    out_specs=pl.BlockSpec((MBLK, HID), lambda i: (i, 0)),
    out_shape=jax.ShapeDtypeStruct((NP, HID), _f32),
)


def _tcF_body(sa_ref, sb_ref, sh_ref, sd_ref, b1_ref, wm_ref, bm_ref,
              hsel_ref, out_ref):
    dis = sd_ref[:, 0:1]
    pre = (sa_ref[...] + sb_ref[...] + sh_ref[...]) * dis + b1_ref[...]
    hsel = jnp.where(pre >= 0, pre, 0.01 * pre)
    hsel_ref[...] = hsel
    z = jnp.dot(hsel, wm_ref[...], preferred_element_type=_f32) + bm_ref[...]
    out_ref[...] = 1.0 / (1.0 + jnp.exp(-z))


_tcF = pl.pallas_call(
    _tcF_body,
    out_shape=[
        jax.ShapeDtypeStruct((N_IDX, HID), _f32),
        jax.ShapeDtypeStruct((N_IDX, 128), _f32),
    ],
)


def kernel(x, edge_index, idx, W0, b0, W1, b1, Wm, bm):
    src = edge_index[0]
    dst = edge_index[1]
    n_pad = E_PAD - N_EDGES
    pad_src = (jnp.arange(n_pad, dtype=jnp.int32) * 97) % N_NODES
    pad_dst = N_NODES + (jnp.arange(n_pad, dtype=jnp.int32) % (NP - N_NODES))
    src2d = jnp.concatenate([src, pad_src]).reshape(E_PAD // CH, CH)
    dst2d = jnp.concatenate([dst, pad_dst]).reshape(E_PAD // CH, CH)

    xp = jnp.pad(x, ((0, NP - N_NODES), (0, 0)))
    zeros_deg = jnp.zeros((NP, DEGW), _f32)
    ones_deg = jnp.ones((CH, DEGW), _f32)
    zeros_hid = jnp.zeros((NP, HID), _f32)
    b0r = b0.reshape(1, HID)
    b1r = b1.reshape(1, HID)
    wm_p = jnp.pad(Wm, ((0, 0), (0, 128 - OUT_CH)))
    bm_p = jnp.pad(bm, (0, 128 - OUT_CH)).reshape(1, 128)

    degp = _deg_kernel(dst2d, ones_deg, zeros_deg)
    ht0, dis16 = _tcA(xp, W0, degp)
    acc0 = _agg_kernel(ht0, src2d, dst2d, zeros_hid)
    ht1 = _tcB(acc0, ht0, dis16, W1, b0r)
    acc1 = _agg_kernel(ht1, src2d, dst2d, zeros_hid)
    sel_a, sel_b, sel_h, sel_d = _sel_kernel(
        acc1[0], acc1[1], ht1, dis16, idx)
    h_sel, out_p = _tcF(sel_a, sel_b, sel_h, sel_d, b1r, wm_p, bm_p)
    return (h_sel, out_p[:, :OUT_CH])


# speedup vs baseline: 41.9436x; 1.3717x over previous
import functools

import jax
import jax.numpy as jnp
from jax import lax
from jax.experimental import pallas as pl
from jax.experimental.pallas import tpu as pltpu
from jax.experimental.pallas import tpu_sc as plsc

N_NODES = 10000
N_EDGES = 320000
IN_CH = 128
HID = 64
OUT_CH = 5
N_IDX = 2048

NC = 2
NS = 16
NW = NC * NS
CH = 128
NP = 10240
ROWS_PER_TILE = NP // NS
CPW = 80
E_PAD = NW * CH * CPW
DEGW = 16
MBLK = 2000
NBUF = 4
DEG_DEPTH = 8
SEL_PER_TILE = N_IDX // NS

_mesh = plsc.VectorSubcoreMesh(core_axis_name="c", subcore_axis_name="s")
_f32 = jnp.float32
_sc_params = pltpu.CompilerParams(use_tc_tiling_on_sc=False)


@functools.partial(
    pl.kernel,
    out_type=jax.ShapeDtypeStruct((NC, NP, DEGW), _f32),
    mesh=_mesh,
    compiler_params=_sc_params,
    scratch_types=[
        pltpu.VMEM((CPW, CH), jnp.int32),
        pltpu.VMEM((CH, DEGW), _f32),
        pltpu.VMEM_SHARED((NP, DEGW), _f32),
        pltpu.SemaphoreType.DMA,
    ],
)
def _deg_kernel(dst_hbm, ones_hbm, zeros_hbm, out_hbm, dst_v, ones_v, acc_sp, sem):
    c = lax.axis_index("c")
    s = lax.axis_index("s")
    wid = c * NS + s
    pltpu.sync_copy(ones_hbm, ones_v)
    pltpu.sync_copy(dst_hbm.at[pl.ds(wid * CPW, CPW)], dst_v)
    row0 = s * ROWS_PER_TILE
    pltpu.sync_copy(zeros_hbm, acc_sp.at[pl.ds(row0, ROWS_PER_TILE)])
    plsc.subcore_barrier()

    for j in range(DEG_DEPTH):
        pltpu.async_copy(ones_v, acc_sp.at[dst_v.at[j]], sem, add=True)

    def body(k, carry):
        pltpu.make_async_copy(ones_v, acc_sp.at[dst_v.at[k]], sem).wait()
        pltpu.async_copy(ones_v, acc_sp.at[dst_v.at[k + DEG_DEPTH]], sem, add=True)
        return carry

    lax.fori_loop(0, CPW - DEG_DEPTH, body, 0)
    for j in range(DEG_DEPTH):
        pltpu.make_async_copy(ones_v, acc_sp.at[dst_v.at[j]], sem).wait()

    plsc.subcore_barrier()
    pltpu.sync_copy(acc_sp.at[pl.ds(row0, ROWS_PER_TILE)],
                    out_hbm.at[c].at[pl.ds(row0, ROWS_PER_TILE)])


def _agg_prologue(ht_hbm, src_hbm, dst_hbm, zeros_hbm,
                  src_v, dst_v, rows, gsems, ssems, acc_sp):
    c = lax.axis_index("c")
    s = lax.axis_index("s")
    wid = c * NS + s
    pltpu.sync_copy(src_hbm.at[pl.ds(wid * CPW, CPW)], src_v)
    pltpu.sync_copy(dst_hbm.at[pl.ds(wid * CPW, CPW)], dst_v)
    row0 = s * ROWS_PER_TILE
    pltpu.sync_copy(zeros_hbm, acc_sp.at[pl.ds(row0, ROWS_PER_TILE)])
    plsc.subcore_barrier()

    for b in range(NBUF):
        pltpu.async_copy(ht_hbm.at[src_v.at[b]], rows[b], gsems[b])

    def body(k, carry):
        j0 = NBUF * k
        for b in range(NBUF):
            pltpu.make_async_copy(ht_hbm.at[src_v.at[j0 + b]],
                                  rows[b], gsems[b]).wait()
            pltpu.async_copy(rows[b], acc_sp.at[dst_v.at[j0 + b]],
                             ssems[b], add=True)
        for b in range(NBUF):
            jn = j0 + NBUF + b

            @pl.when(jn < CPW)
            def _(b=b, jn=jn):
                pltpu.make_async_copy(rows[b], acc_sp.at[dst_v.at[jn]],
                                      ssems[b]).wait()
                pltpu.async_copy(ht_hbm.at[src_v.at[jn]], rows[b], gsems[b])

        return carry

    lax.fori_loop(0, CPW // NBUF, body, 0)
    for b in range(NBUF):
        pltpu.make_async_copy(rows[b], acc_sp.at[dst_v.at[b]], ssems[b]).wait()

    plsc.subcore_barrier()
    return c, s, row0


@functools.partial(
    pl.kernel,
    out_type=jax.ShapeDtypeStruct((NC, NP, HID), _f32),
    mesh=_mesh,
    compiler_params=_sc_params,
    scratch_types=[
        pltpu.VMEM((CPW, CH), jnp.int32),
        pltpu.VMEM((CPW, CH), jnp.int32),
    ] + [pltpu.VMEM((CH, HID), _f32)] * NBUF
      + [pltpu.SemaphoreType.DMA] * (2 * NBUF)
      + [pltpu.VMEM_SHARED((NP, HID), _f32)],
)
def _agg_kernel(ht_hbm, src_hbm, dst_hbm, zeros_hbm, out_hbm,
                src_v, dst_v, *bufs):
    rows = bufs[:NBUF]
    gsems = bufs[NBUF:2 * NBUF]
    ssems = bufs[2 * NBUF:3 * NBUF]
    acc_sp = bufs[3 * NBUF]
    c, s, row0 = _agg_prologue(ht_hbm, src_hbm, dst_hbm, zeros_hbm,
                               src_v, dst_v, rows, gsems, ssems, acc_sp)
    pltpu.sync_copy(acc_sp.at[pl.ds(row0, ROWS_PER_TILE)],
                    out_hbm.at[c].at[pl.ds(row0, ROWS_PER_TILE)])


@functools.partial(
    pl.kernel,
    out_type=(
        jax.ShapeDtypeStruct((NC, NP, HID), _f32),
        jax.ShapeDtypeStruct((NC, N_IDX, HID), _f32),
        jax.ShapeDtypeStruct((N_IDX, HID), _f32),
        jax.ShapeDtypeStruct((N_IDX, DEGW), _f32),
    ),
    mesh=_mesh,
    compiler_params=_sc_params,
    scratch_types=[
        pltpu.VMEM((CPW, CH), jnp.int32),
        pltpu.VMEM((CPW, CH), jnp.int32),
    ] + [pltpu.VMEM((CH, HID), _f32)] * NBUF
      + [pltpu.SemaphoreType.DMA] * (2 * NBUF)
      + [
        pltpu.VMEM((SEL_PER_TILE,), jnp.int32),
        pltpu.VMEM((SEL_PER_TILE, HID), _f32),
        pltpu.VMEM((SEL_PER_TILE, DEGW), _f32),
        pltpu.VMEM_SHARED((NP, HID), _f32),
    ],
)
def _agg_sel_kernel(ht_hbm, src_hbm, dst_hbm, zeros_hbm, dis_hbm, idx_hbm,
                    out_hbm, selacc_hbm, selht_hbm, seldis_hbm,
                    src_v, dst_v, *bufs):
    rows = bufs[:NBUF]
    gsems = bufs[NBUF:2 * NBUF]
    ssems = bufs[2 * NBUF:3 * NBUF]
    idxsel_v, selrow_v, seldis_v, acc_sp = bufs[3 * NBUF:]

    c, s, row0 = _agg_prologue(ht_hbm, src_hbm, dst_hbm, zeros_hbm,
                               src_v, dst_v, rows, gsems, ssems, acc_sp)
    pltpu.sync_copy(acc_sp.at[pl.ds(row0, ROWS_PER_TILE)],
                    out_hbm.at[c].at[pl.ds(row0, ROWS_PER_TILE)])

    base = s * SEL_PER_TILE
    pltpu.sync_copy(idx_hbm.at[pl.ds(base, SEL_PER_TILE)], idxsel_v)

    @pl.when(c == 0)
    def _():
        pltpu.async_copy(ht_hbm.at[idxsel_v], selrow_v, gsems[0]).wait()
        pltpu.sync_copy(selrow_v, selht_hbm.at[pl.ds(base, SEL_PER_TILE)])

    @pl.when(c == 1)
    def _():
        pltpu.async_copy(dis_hbm.at[idxsel_v], seldis_v, gsems[0]).wait()
        pltpu.sync_copy(seldis_v, seldis_hbm.at[pl.ds(base, SEL_PER_TILE)])

    plsc.subcore_barrier()
    pltpu.async_copy(out_hbm.at[c].at[idxsel_v], selrow_v, gsems[1]).wait()
    pltpu.sync_copy(selrow_v,
                    selacc_hbm.at[c].at[pl.ds(base, SEL_PER_TILE)])


def _tcA_body(x_ref, w0_ref, degp_ref, ht_ref, dis_ref):
    deg = degp_ref[0] + degp_ref[1] + 1.0
    dis = lax.rsqrt(deg)
    h = jnp.dot(x_ref[...], w0_ref[...], preferred_element_type=_f32)
    ht_ref[...] = h * dis[:, 0:1]
    dis_ref[...] = dis


_tcA = pl.pallas_call(
    _tcA_body,
    grid=(N_NODES // MBLK,),
    in_specs=[
        pl.BlockSpec((MBLK, IN_CH), lambda i: (i, 0)),
        pl.BlockSpec((IN_CH, HID), lambda i: (0, 0)),
        pl.BlockSpec((NC, MBLK, DEGW), lambda i: (0, i, 0)),
    ],
    out_specs=[
        pl.BlockSpec((MBLK, HID), lambda i: (i, 0)),
        pl.BlockSpec((MBLK, DEGW), lambda i: (i, 0)),
    ],
    out_shape=[
        jax.ShapeDtypeStruct((NP, HID), _f32),
        jax.ShapeDtypeStruct((NP, DEGW), _f32),
    ],
)


def _tcB_body(accp_ref, ht0_ref, dis_ref, w1_ref, b0_ref, ht1_ref):
    dis = dis_ref[:, 0:1]
    pre = (accp_ref[0] + accp_ref[1] + ht0_ref[...]) * dis + b0_ref[...]
    h1 = jnp.where(pre >= 0, pre, 0.01 * pre)
    ht1_ref[...] = jnp.dot(h1, w1_ref[...], preferred_element_type=_f32) * dis


_tcB = pl.pallas_call(
    _tcB_body,
    grid=(N_NODES // MBLK,),
    in_specs=[
        pl.BlockSpec((NC, MBLK, HID), lambda i: (0, i, 0)),
        pl.BlockSpec((MBLK, HID), lambda i: (i, 0)),
        pl.BlockSpec((MBLK, DEGW), lambda i: (i, 0)),
        pl.BlockSpec((HID, HID), lambda i: (0, 0)),
        pl.BlockSpec((1, HID), lambda i: (0, 0)),
    ],
    out_specs=pl.BlockSpec((MBLK, HID), lambda i: (i, 0)),
    out_shape=jax.ShapeDtypeStruct((NP, HID), _f32),
)


def _tcF_body(sacc_ref, sh_ref, sd_ref, b1_ref, wm_ref, bm_ref,
              hsel_ref, out_ref):
    dis = sd_ref[:, 0:1]
    pre = (sacc_ref[0] + sacc_ref[1] + sh_ref[...]) * dis + b1_ref[...]
    hsel = jnp.where(pre >= 0, pre, 0.01 * pre)
    hsel_ref[...] = hsel
    z = jnp.dot(hsel, wm_ref[...], preferred_element_type=_f32) + bm_ref[...]
    out_ref[...] = 1.0 / (1.0 + jnp.exp(-z))


_tcF = pl.pallas_call(
    _tcF_body,
    out_shape=[
        jax.ShapeDtypeStruct((N_IDX, HID), _f32),
        jax.ShapeDtypeStruct((N_IDX, 128), _f32),
    ],
)


def kernel(x, edge_index, idx, W0, b0, W1, b1, Wm, bm):
    src = edge_index[0]
    dst = edge_index[1]
    n_pad = E_PAD - N_EDGES
    pad_src = (jnp.arange(n_pad, dtype=jnp.int32) * 97) % N_NODES
    pad_dst = N_NODES + (jnp.arange(n_pad, dtype=jnp.int32) % (NP - N_NODES))
    src2d = jnp.concatenate([src, pad_src]).reshape(E_PAD // CH, CH)
    dst2d = jnp.concatenate([dst, pad_dst]).reshape(E_PAD // CH, CH)

    zeros_deg = jnp.zeros((ROWS_PER_TILE, DEGW), _f32)
    ones_deg = jnp.ones((CH, DEGW), _f32)
    zeros_hid = jnp.zeros((ROWS_PER_TILE, HID), _f32)
    b0r = b0.reshape(1, HID)
    b1r = b1.reshape(1, HID)
    wm_p = jnp.pad(Wm, ((0, 0), (0, 128 - OUT_CH)))
    bm_p = jnp.pad(bm, (0, 128 - OUT_CH)).reshape(1, 128)

    degp = _deg_kernel(dst2d, ones_deg, zeros_deg)
    ht0, dis16 = _tcA(x, W0, degp)
    acc0 = _agg_kernel(ht0, src2d, dst2d, zeros_hid)
    ht1 = _tcB(acc0, ht0, dis16, W1, b0r)
    _, sel_acc, sel_ht, sel_dis = _agg_sel_kernel(
        ht1, src2d, dst2d, zeros_hid, dis16, idx)
    h_sel, out_p = _tcF(sel_acc, sel_ht, sel_dis, b1r, wm_p, bm_p)
    return (h_sel, out_p[:, :OUT_CH])


# speedup vs baseline: 42.2026x; 1.0062x over previous
import functools

import jax
import jax.numpy as jnp
from jax import lax
from jax.experimental import pallas as pl
from jax.experimental.pallas import tpu as pltpu
from jax.experimental.pallas import tpu_sc as plsc

N_NODES = 10000
N_EDGES = 320000
IN_CH = 128
HID = 64
OUT_CH = 5
N_IDX = 2048

NC = 2
NS = 16
NW = NC * NS
CH = 128
NP = 10240
ROWS_PER_TILE = NP // NS
CPW = 80
E_PAD = NW * CH * CPW
DEGW = 16
MBLK = 2000
NBUF = 5
DEG_DEPTH = 8
SEL_PER_TILE = N_IDX // NS

_mesh = plsc.VectorSubcoreMesh(core_axis_name="c", subcore_axis_name="s")
_f32 = jnp.float32
_sc_params = pltpu.CompilerParams(use_tc_tiling_on_sc=False)


@functools.partial(
    pl.kernel,
    out_type=jax.ShapeDtypeStruct((NC, NP, DEGW), _f32),
    mesh=_mesh,
    compiler_params=_sc_params,
    scratch_types=[
        pltpu.VMEM((CPW, CH), jnp.int32),
        pltpu.VMEM((CH, DEGW), _f32),
        pltpu.VMEM_SHARED((NP, DEGW), _f32),
        pltpu.SemaphoreType.DMA,
    ],
)
def _deg_kernel(dst_hbm, ones_hbm, zeros_hbm, out_hbm, dst_v, ones_v, acc_sp, sem):
    c = lax.axis_index("c")
    s = lax.axis_index("s")
    wid = c * NS + s
    pltpu.sync_copy(ones_hbm, ones_v)
    pltpu.sync_copy(dst_hbm.at[pl.ds(wid * CPW, CPW)], dst_v)
    row0 = s * ROWS_PER_TILE
    pltpu.sync_copy(zeros_hbm, acc_sp.at[pl.ds(row0, ROWS_PER_TILE)])
    plsc.subcore_barrier()

    for j in range(DEG_DEPTH):
        pltpu.async_copy(ones_v, acc_sp.at[dst_v.at[j]], sem, add=True)

    def body(k, carry):
        pltpu.make_async_copy(ones_v, acc_sp.at[dst_v.at[k]], sem).wait()
        pltpu.async_copy(ones_v, acc_sp.at[dst_v.at[k + DEG_DEPTH]], sem, add=True)
        return carry

    lax.fori_loop(0, CPW - DEG_DEPTH, body, 0)
    for j in range(DEG_DEPTH):
        pltpu.make_async_copy(ones_v, acc_sp.at[dst_v.at[j]], sem).wait()

    plsc.subcore_barrier()
    pltpu.sync_copy(acc_sp.at[pl.ds(row0, ROWS_PER_TILE)],
                    out_hbm.at[c].at[pl.ds(row0, ROWS_PER_TILE)])


def _agg_prologue(ht_hbm, src_hbm, dst_hbm, zeros_hbm,
                  src_v, dst_v, rows, gsems, ssems, acc_sp):
    c = lax.axis_index("c")
    s = lax.axis_index("s")
    wid = c * NS + s
    pltpu.sync_copy(src_hbm.at[pl.ds(wid * CPW, CPW)], src_v)
    pltpu.sync_copy(dst_hbm.at[pl.ds(wid * CPW, CPW)], dst_v)
    row0 = s * ROWS_PER_TILE
    pltpu.sync_copy(zeros_hbm, acc_sp.at[pl.ds(row0, ROWS_PER_TILE)])
    plsc.subcore_barrier()

    for b in range(NBUF):
        pltpu.async_copy(ht_hbm.at[src_v.at[b]], rows[b], gsems[b])

    def body(k, carry):
        j0 = NBUF * k
        for b in range(NBUF):
            pltpu.make_async_copy(ht_hbm.at[src_v.at[j0 + b]],
                                  rows[b], gsems[b]).wait()
            pltpu.async_copy(rows[b], acc_sp.at[dst_v.at[j0 + b]],
                             ssems[b], add=True)
        for b in range(NBUF):
            jn = j0 + NBUF + b

            @pl.when(jn < CPW)
            def _(b=b, jn=jn):
                pltpu.make_async_copy(rows[b], acc_sp.at[dst_v.at[jn]],
                                      ssems[b]).wait()
                pltpu.async_copy(ht_hbm.at[src_v.at[jn]], rows[b], gsems[b])

        return carry

    lax.fori_loop(0, CPW // NBUF, body, 0)
    for b in range(NBUF):
        pltpu.make_async_copy(rows[b], acc_sp.at[dst_v.at[b]], ssems[b]).wait()

    plsc.subcore_barrier()
    return c, s, row0


@functools.partial(
    pl.kernel,
    out_type=jax.ShapeDtypeStruct((NC, NP, HID), _f32),
    mesh=_mesh,
    compiler_params=_sc_params,
    scratch_types=[
        pltpu.VMEM((CPW, CH), jnp.int32),
        pltpu.VMEM((CPW, CH), jnp.int32),
    ] + [pltpu.VMEM((CH, HID), _f32)] * NBUF
      + [pltpu.SemaphoreType.DMA] * (2 * NBUF)
      + [pltpu.VMEM_SHARED((NP, HID), _f32)],
)
def _agg_kernel(ht_hbm, src_hbm, dst_hbm, zeros_hbm, out_hbm,
                src_v, dst_v, *bufs):
    rows = bufs[:NBUF]
    gsems = bufs[NBUF:2 * NBUF]
    ssems = bufs[2 * NBUF:3 * NBUF]
    acc_sp = bufs[3 * NBUF]
    c, s, row0 = _agg_prologue(ht_hbm, src_hbm, dst_hbm, zeros_hbm,
                               src_v, dst_v, rows, gsems, ssems, acc_sp)
    pltpu.sync_copy(acc_sp.at[pl.ds(row0, ROWS_PER_TILE)],
                    out_hbm.at[c].at[pl.ds(row0, ROWS_PER_TILE)])


@functools.partial(
    pl.kernel,
    out_type=(
        jax.ShapeDtypeStruct((NC, NP, HID), _f32),
        jax.ShapeDtypeStruct((NC, N_IDX, HID), _f32),
        jax.ShapeDtypeStruct((N_IDX, HID), _f32),
        jax.ShapeDtypeStruct((N_IDX, DEGW), _f32),
    ),
    mesh=_mesh,
    compiler_params=_sc_params,
    scratch_types=[
        pltpu.VMEM((CPW, CH), jnp.int32),
        pltpu.VMEM((CPW, CH), jnp.int32),
    ] + [pltpu.VMEM((CH, HID), _f32)] * NBUF
      + [pltpu.SemaphoreType.DMA] * (2 * NBUF)
      + [
        pltpu.VMEM((SEL_PER_TILE,), jnp.int32),
        pltpu.VMEM((SEL_PER_TILE, HID), _f32),
        pltpu.VMEM((SEL_PER_TILE, DEGW), _f32),
        pltpu.VMEM_SHARED((NP, HID), _f32),
    ],
)
def _agg_sel_kernel(ht_hbm, src_hbm, dst_hbm, zeros_hbm, dis_hbm, idx_hbm,
                    out_hbm, selacc_hbm, selht_hbm, seldis_hbm,
                    src_v, dst_v, *bufs):
    rows = bufs[:NBUF]
    gsems = bufs[NBUF:2 * NBUF]
    ssems = bufs[2 * NBUF:3 * NBUF]
    idxsel_v, selrow_v, seldis_v, acc_sp = bufs[3 * NBUF:]

    c, s, row0 = _agg_prologue(ht_hbm, src_hbm, dst_hbm, zeros_hbm,
                               src_v, dst_v, rows, gsems, ssems, acc_sp)
    pltpu.sync_copy(acc_sp.at[pl.ds(row0, ROWS_PER_TILE)],
                    out_hbm.at[c].at[pl.ds(row0, ROWS_PER_TILE)])

    base = s * SEL_PER_TILE
    pltpu.sync_copy(idx_hbm.at[pl.ds(base, SEL_PER_TILE)], idxsel_v)

    @pl.when(c == 0)
    def _():
        pltpu.async_copy(ht_hbm.at[idxsel_v], selrow_v, gsems[0]).wait()
        pltpu.sync_copy(selrow_v, selht_hbm.at[pl.ds(base, SEL_PER_TILE)])

    @pl.when(c == 1)
    def _():
        pltpu.async_copy(dis_hbm.at[idxsel_v], seldis_v, gsems[0]).wait()
        pltpu.sync_copy(seldis_v, seldis_hbm.at[pl.ds(base, SEL_PER_TILE)])

    plsc.subcore_barrier()
    pltpu.async_copy(out_hbm.at[c].at[idxsel_v], selrow_v, gsems[1]).wait()
    pltpu.sync_copy(selrow_v,
                    selacc_hbm.at[c].at[pl.ds(base, SEL_PER_TILE)])


def _tcA_body(x_ref, w0_ref, degp_ref, ht_ref, dis_ref):
    deg = degp_ref[0] + degp_ref[1] + 1.0
    dis = lax.rsqrt(deg)
    h = jnp.dot(x_ref[...], w0_ref[...], preferred_element_type=_f32)
    ht_ref[...] = h * dis[:, 0:1]
    dis_ref[...] = dis


_tcA = pl.pallas_call(
    _tcA_body,
    grid=(N_NODES // MBLK,),
    in_specs=[
        pl.BlockSpec((MBLK, IN_CH), lambda i: (i, 0)),
        pl.BlockSpec((IN_CH, HID), lambda i: (0, 0)),
        pl.BlockSpec((NC, MBLK, DEGW), lambda i: (0, i, 0)),
    ],
    out_specs=[
        pl.BlockSpec((MBLK, HID), lambda i: (i, 0)),
        pl.BlockSpec((MBLK, DEGW), lambda i: (i, 0)),
    ],
    out_shape=[
        jax.ShapeDtypeStruct((NP, HID), _f32),
        jax.ShapeDtypeStruct((NP, DEGW), _f32),
    ],
)


def _tcB_body(accp_ref, ht0_ref, dis_ref, w1_ref, b0_ref, ht1_ref):
    dis = dis_ref[:, 0:1]
    pre = (accp_ref[0] + accp_ref[1] + ht0_ref[...]) * dis + b0_ref[...]
    h1 = jnp.where(pre >= 0, pre, 0.01 * pre)
    ht1_ref[...] = jnp.dot(h1, w1_ref[...], preferred_element_type=_f32) * dis


_tcB = pl.pallas_call(
    _tcB_body,
    grid=(N_NODES // MBLK,),
    in_specs=[
        pl.BlockSpec((NC, MBLK, HID), lambda i: (0, i, 0)),
        pl.BlockSpec((MBLK, HID), lambda i: (i, 0)),
        pl.BlockSpec((MBLK, DEGW), lambda i: (i, 0)),
        pl.BlockSpec((HID, HID), lambda i: (0, 0)),
        pl.BlockSpec((1, HID), lambda i: (0, 0)),
    ],
    out_specs=pl.BlockSpec((MBLK, HID), lambda i: (i, 0)),
    out_shape=jax.ShapeDtypeStruct((NP, HID), _f32),
)


def _tcF_body(sacc_ref, sh_ref, sd_ref, b1_ref, wm_ref, bm_ref,
              hsel_ref, out_ref):
    dis = sd_ref[:, 0:1]
    pre = (sacc_ref[0] + sacc_ref[1] + sh_ref[...]) * dis + b1_ref[...]
    hsel = jnp.where(pre >= 0, pre, 0.01 * pre)
    hsel_ref[...] = hsel
    z = jnp.dot(hsel, wm_ref[...], preferred_element_type=_f32) + bm_ref[...]
    out_ref[...] = 1.0 / (1.0 + jnp.exp(-z))


_tcF = pl.pallas_call(
    _tcF_body,
    out_shape=[
        jax.ShapeDtypeStruct((N_IDX, HID), _f32),
        jax.ShapeDtypeStruct((N_IDX, 128), _f32),
    ],
)


def kernel(x, edge_index, idx, W0, b0, W1, b1, Wm, bm):
    src = edge_index[0]
    dst = edge_index[1]
    n_pad = E_PAD - N_EDGES
    pad_src = (jnp.arange(n_pad, dtype=jnp.int32) * 97) % N_NODES
    pad_dst = N_NODES + (jnp.arange(n_pad, dtype=jnp.int32) % (NP - N_NODES))
    src2d = jnp.concatenate([src, pad_src]).reshape(E_PAD // CH, CH)
    dst2d = jnp.concatenate([dst, pad_dst]).reshape(E_PAD // CH, CH)

    zeros_deg = jnp.zeros((ROWS_PER_TILE, DEGW), _f32)
    ones_deg = jnp.ones((CH, DEGW), _f32)
    zeros_hid = jnp.zeros((ROWS_PER_TILE, HID), _f32)
    b0r = b0.reshape(1, HID)
    b1r = b1.reshape(1, HID)
    wm_p = jnp.pad(Wm, ((0, 0), (0, 128 - OUT_CH)))
    bm_p = jnp.pad(bm, (0, 128 - OUT_CH)).reshape(1, 128)

    degp = _deg_kernel(dst2d, ones_deg, zeros_deg)
    ht0, dis16 = _tcA(x, W0, degp)
    acc0 = _agg_kernel(ht0, src2d, dst2d, zeros_hid)
    ht1 = _tcB(acc0, ht0, dis16, W1, b0r)
    _, sel_acc, sel_ht, sel_dis = _agg_sel_kernel(
        ht1, src2d, dst2d, zeros_hid, dis16, idx)
    h_sel, out_p = _tcF(sel_acc, sel_ht, sel_dis, b1r, wm_p, bm_p)
    return (h_sel, out_p[:, :OUT_CH])


# speedup vs baseline: 42.6334x; 1.0102x over previous
import functools

import jax
import jax.numpy as jnp
from jax import lax
from jax.experimental import pallas as pl
from jax.experimental.pallas import tpu as pltpu
from jax.experimental.pallas import tpu_sc as plsc

N_NODES = 10000
N_EDGES = 320000
IN_CH = 128
HID = 64
OUT_CH = 5
N_IDX = 2048

NC = 2
NS = 16
NW = NC * NS
CH = 128
NP = 10240
ROWS_PER_TILE = NP // NS
CPW = 80
E_PAD = NW * CH * CPW
DEGW = 8
MBLK = 2000
NBUF = 5
DEG_DEPTH = 8
SEL_PER_TILE = N_IDX // NS

_mesh = plsc.VectorSubcoreMesh(core_axis_name="c", subcore_axis_name="s")
_f32 = jnp.float32
_sc_params = pltpu.CompilerParams(use_tc_tiling_on_sc=False)


@functools.partial(
    pl.kernel,
    out_type=jax.ShapeDtypeStruct((NC, NP, DEGW), _f32),
    mesh=_mesh,
    compiler_params=_sc_params,
    scratch_types=[
        pltpu.VMEM((CPW, CH), jnp.int32),
        pltpu.VMEM((CH, DEGW), _f32),
        pltpu.VMEM_SHARED((NP, DEGW), _f32),
        pltpu.SemaphoreType.DMA,
    ],
)
def _deg_kernel(dst_hbm, ones_hbm, zeros_hbm, out_hbm, dst_v, ones_v, acc_sp, sem):
    c = lax.axis_index("c")
    s = lax.axis_index("s")
    wid = c * NS + s
    pltpu.sync_copy(ones_hbm, ones_v)
    pltpu.sync_copy(dst_hbm.at[pl.ds(wid * CPW, CPW)], dst_v)
    row0 = s * ROWS_PER_TILE
    pltpu.sync_copy(zeros_hbm, acc_sp.at[pl.ds(row0, ROWS_PER_TILE)])
    plsc.subcore_barrier()

    for j in range(DEG_DEPTH):
        pltpu.async_copy(ones_v, acc_sp.at[dst_v.at[j]], sem, add=True)

    def body(k, carry):
        pltpu.make_async_copy(ones_v, acc_sp.at[dst_v.at[k]], sem).wait()
        pltpu.async_copy(ones_v, acc_sp.at[dst_v.at[k + DEG_DEPTH]], sem, add=True)
        return carry

    lax.fori_loop(0, CPW - DEG_DEPTH, body, 0)
    for j in range(DEG_DEPTH):
        pltpu.make_async_copy(ones_v, acc_sp.at[dst_v.at[j]], sem).wait()

    plsc.subcore_barrier()
    pltpu.sync_copy(acc_sp.at[pl.ds(row0, ROWS_PER_TILE)],
                    out_hbm.at[c].at[pl.ds(row0, ROWS_PER_TILE)])


def _agg_prologue(ht_hbm, src_hbm, dst_hbm, zeros_hbm,
                  src_v, dst_v, rows, gsems, ssems, acc_sp):
    c = lax.axis_index("c")
    s = lax.axis_index("s")
    wid = c * NS + s
    pltpu.sync_copy(src_hbm.at[pl.ds(wid * CPW, CPW)], src_v)
    pltpu.sync_copy(dst_hbm.at[pl.ds(wid * CPW, CPW)], dst_v)
    row0 = s * ROWS_PER_TILE
    pltpu.sync_copy(zeros_hbm, acc_sp.at[pl.ds(row0, ROWS_PER_TILE)])
    plsc.subcore_barrier()

    for b in range(NBUF):
        pltpu.async_copy(ht_hbm.at[src_v.at[b]], rows[b], gsems[b])

    def body(k, carry):
        j0 = NBUF * k
        for b in range(NBUF):
            pltpu.make_async_copy(ht_hbm.at[src_v.at[j0 + b]],
                                  rows[b], gsems[b]).wait()
            pltpu.async_copy(rows[b], acc_sp.at[dst_v.at[j0 + b]],
                             ssems[b], add=True)
        for b in range(NBUF):
            jn = j0 + NBUF + b

            @pl.when(jn < CPW)
            def _(b=b, jn=jn):
                pltpu.make_async_copy(rows[b], acc_sp.at[dst_v.at[jn]],
                                      ssems[b]).wait()
                pltpu.async_copy(ht_hbm.at[src_v.at[jn]], rows[b], gsems[b])

        return carry

    lax.fori_loop(0, CPW // NBUF, body, 0)
    for b in range(NBUF):
        pltpu.make_async_copy(rows[b], acc_sp.at[dst_v.at[b]], ssems[b]).wait()

    plsc.subcore_barrier()
    return c, s, row0


@functools.partial(
    pl.kernel,
    out_type=jax.ShapeDtypeStruct((NC, NP, HID), _f32),
    mesh=_mesh,
    compiler_params=_sc_params,
    scratch_types=[
        pltpu.VMEM((CPW, CH), jnp.int32),
        pltpu.VMEM((CPW, CH), jnp.int32),
    ] + [pltpu.VMEM((CH, HID), _f32)] * NBUF
      + [pltpu.SemaphoreType.DMA] * (2 * NBUF)
      + [pltpu.VMEM_SHARED((NP, HID), _f32)],
)
def _agg_kernel(ht_hbm, src_hbm, dst_hbm, zeros_hbm, out_hbm,
                src_v, dst_v, *bufs):
    rows = bufs[:NBUF]
    gsems = bufs[NBUF:2 * NBUF]
    ssems = bufs[2 * NBUF:3 * NBUF]
    acc_sp = bufs[3 * NBUF]
    c, s, row0 = _agg_prologue(ht_hbm, src_hbm, dst_hbm, zeros_hbm,
                               src_v, dst_v, rows, gsems, ssems, acc_sp)
    pltpu.sync_copy(acc_sp.at[pl.ds(row0, ROWS_PER_TILE)],
                    out_hbm.at[c].at[pl.ds(row0, ROWS_PER_TILE)])


@functools.partial(
    pl.kernel,
    out_type=(
        jax.ShapeDtypeStruct((NC, NP, HID), _f32),
        jax.ShapeDtypeStruct((NC, N_IDX, HID), _f32),
        jax.ShapeDtypeStruct((N_IDX, HID), _f32),
        jax.ShapeDtypeStruct((N_IDX, DEGW), _f32),
    ),
    mesh=_mesh,
    compiler_params=_sc_params,
    scratch_types=[
        pltpu.VMEM((CPW, CH), jnp.int32),
        pltpu.VMEM((CPW, CH), jnp.int32),
    ] + [pltpu.VMEM((CH, HID), _f32)] * NBUF
      + [pltpu.SemaphoreType.DMA] * (2 * NBUF)
      + [
        pltpu.VMEM((SEL_PER_TILE,), jnp.int32),
        pltpu.VMEM((SEL_PER_TILE, HID), _f32),
        pltpu.VMEM((SEL_PER_TILE, DEGW), _f32),
        pltpu.VMEM_SHARED((NP, HID), _f32),
    ],
)
def _agg_sel_kernel(ht_hbm, src_hbm, dst_hbm, zeros_hbm, dis_hbm, idx_hbm,
                    out_hbm, selacc_hbm, selht_hbm, seldis_hbm,
                    src_v, dst_v, *bufs):
    rows = bufs[:NBUF]
    gsems = bufs[NBUF:2 * NBUF]
    ssems = bufs[2 * NBUF:3 * NBUF]
    idxsel_v, selrow_v, seldis_v, acc_sp = bufs[3 * NBUF:]

    c, s, row0 = _agg_prologue(ht_hbm, src_hbm, dst_hbm, zeros_hbm,
                               src_v, dst_v, rows, gsems, ssems, acc_sp)
    pltpu.sync_copy(acc_sp.at[pl.ds(row0, ROWS_PER_TILE)],
                    out_hbm.at[c].at[pl.ds(row0, ROWS_PER_TILE)])

    base = s * SEL_PER_TILE
    pltpu.sync_copy(idx_hbm.at[pl.ds(base, SEL_PER_TILE)], idxsel_v)

    @pl.when(c == 0)
    def _():
        pltpu.async_copy(ht_hbm.at[idxsel_v], selrow_v, gsems[0]).wait()
        pltpu.sync_copy(selrow_v, selht_hbm.at[pl.ds(base, SEL_PER_TILE)])

    @pl.when(c == 1)
    def _():
        pltpu.async_copy(dis_hbm.at[idxsel_v], seldis_v, gsems[0]).wait()
        pltpu.sync_copy(seldis_v, seldis_hbm.at[pl.ds(base, SEL_PER_TILE)])

    plsc.subcore_barrier()
    pltpu.async_copy(out_hbm.at[c].at[idxsel_v], selrow_v, gsems[1]).wait()
    pltpu.sync_copy(selrow_v,
                    selacc_hbm.at[c].at[pl.ds(base, SEL_PER_TILE)])


def _tcMM_body(x_ref, w0_ref, h_ref):
    h_ref[...] = jnp.dot(x_ref[...], w0_ref[...], preferred_element_type=_f32)


_tcMM = pl.pallas_call(
    _tcMM_body,
    grid=(N_NODES // MBLK,),
    in_specs=[
        pl.BlockSpec((MBLK, IN_CH), lambda i: (i, 0)),
        pl.BlockSpec((IN_CH, HID), lambda i: (0, 0)),
    ],
    out_specs=pl.BlockSpec((MBLK, HID), lambda i: (i, 0)),
    out_shape=jax.ShapeDtypeStruct((NP, HID), _f32),
)


def _tcScale_body(h_ref, degp_ref, ht_ref, dis_ref):
    deg = degp_ref[0] + degp_ref[1] + 1.0
    dis = lax.rsqrt(deg)
    ht_ref[...] = h_ref[...] * dis[:, 0:1]
    dis_ref[...] = dis


_tcScale = pl.pallas_call(
    _tcScale_body,
    grid=(N_NODES // MBLK,),
    in_specs=[
        pl.BlockSpec((MBLK, HID), lambda i: (i, 0)),
        pl.BlockSpec((NC, MBLK, DEGW), lambda i: (0, i, 0)),
    ],
    out_specs=[
        pl.BlockSpec((MBLK, HID), lambda i: (i, 0)),
        pl.BlockSpec((MBLK, DEGW), lambda i: (i, 0)),
    ],
    out_shape=[
        jax.ShapeDtypeStruct((NP, HID), _f32),
        jax.ShapeDtypeStruct((NP, DEGW), _f32),
    ],
)


def _tcB_body(accp_ref, ht0_ref, dis_ref, w1_ref, b0_ref, ht1_ref):
    dis = dis_ref[:, 0:1]
    pre = (accp_ref[0] + accp_ref[1] + ht0_ref[...]) * dis + b0_ref[...]
    h1 = jnp.where(pre >= 0, pre, 0.01 * pre)
    ht1_ref[...] = jnp.dot(h1, w1_ref[...], preferred_element_type=_f32) * dis


_tcB = pl.pallas_call(
    _tcB_body,
    grid=(N_NODES // MBLK,),
    in_specs=[
        pl.BlockSpec((NC, MBLK, HID), lambda i: (0, i, 0)),
        pl.BlockSpec((MBLK, HID), lambda i: (i, 0)),
        pl.BlockSpec((MBLK, DEGW), lambda i: (i, 0)),
        pl.BlockSpec((HID, HID), lambda i: (0, 0)),
        pl.BlockSpec((1, HID), lambda i: (0, 0)),
    ],
    out_specs=pl.BlockSpec((MBLK, HID), lambda i: (i, 0)),
    out_shape=jax.ShapeDtypeStruct((NP, HID), _f32),
)


def _tcF_body(sacc_ref, sh_ref, sd_ref, b1_ref, wm_ref, bm_ref,
              hsel_ref, out_ref):
    dis = sd_ref[:, 0:1]
    pre = (sacc_ref[0] + sacc_ref[1] + sh_ref[...]) * dis + b1_ref[...]
    hsel = jnp.where(pre >= 0, pre, 0.01 * pre)
    hsel_ref[...] = hsel
    z = jnp.dot(hsel, wm_ref[...], preferred_element_type=_f32) + bm_ref[...]
    out_ref[...] = 1.0 / (1.0 + jnp.exp(-z))


_tcF = pl.pallas_call(
    _tcF_body,
    out_shape=[
        jax.ShapeDtypeStruct((N_IDX, HID), _f32),
        jax.ShapeDtypeStruct((N_IDX, 128), _f32),
    ],
)


def kernel(x, edge_index, idx, W0, b0, W1, b1, Wm, bm):
    src = edge_index[0]
    dst = edge_index[1]
    n_pad = E_PAD - N_EDGES
    pad_src = (jnp.arange(n_pad, dtype=jnp.int32) * 97) % N_NODES
    pad_dst = N_NODES + (jnp.arange(n_pad, dtype=jnp.int32) % (NP - N_NODES))
    src2d = jnp.concatenate([src, pad_src]).reshape(E_PAD // CH, CH)
    dst2d = jnp.concatenate([dst, pad_dst]).reshape(E_PAD // CH, CH)

    zeros_deg = jnp.zeros((ROWS_PER_TILE, DEGW), _f32)
    ones_deg = jnp.ones((CH, DEGW), _f32)
    zeros_hid = jnp.zeros((ROWS_PER_TILE, HID), _f32)
    b0r = b0.reshape(1, HID)
    b1r = b1.reshape(1, HID)
    wm_p = jnp.pad(Wm, ((0, 0), (0, 128 - OUT_CH)))
    bm_p = jnp.pad(bm, (0, 128 - OUT_CH)).reshape(1, 128)

    h0 = _tcMM(x, W0)
    degp = _deg_kernel(dst2d, ones_deg, zeros_deg)
    ht0, dis16 = _tcScale(h0, degp)
    acc0 = _agg_kernel(ht0, src2d, dst2d, zeros_hid)
    ht1 = _tcB(acc0, ht0, dis16, W1, b0r)
    _, sel_acc, sel_ht, sel_dis = _agg_sel_kernel(
        ht1, src2d, dst2d, zeros_hid, dis16, idx)
    h_sel, out_p = _tcF(sel_acc, sel_ht, sel_dis, b1r, wm_p, bm_p)
    return (h_sel, out_p[:, :OUT_CH])
